# Initial kernel scaffold; baseline (speedup 1.0000x reference)
import jax
import jax.numpy as jnp
from jax.experimental import pallas as pl


def kernel(pos, batch, W1, g1, b1, W2, g2, b2, W3, g3, b3, Wf, gf, bf, Wc1, bc1, gcl1, bcl1, Wc2, bc2, gcl2, bcl2, Wc3, bc3):
    raise NotImplementedError("write your pallas kernel here")


# speedup vs baseline: 2.7311x; 2.7311x over previous
import functools

import jax
import jax.numpy as jnp
from jax import lax
from jax.experimental import pallas as pl
from jax.experimental.pallas import tpu as pltpu
from jax.experimental.pallas import tpu_sc as plsc

N = 8192
K = 20
NSEG = 8
NF = float(N)
INF = jnp.inf
CP = 128


def _pad_sq_body(x_ref, xp_ref, sq_ref):
    x = x_ref[...]
    c = x.shape[1]
    if c == CP:
        xp_ref[...] = x
    else:
        xp_ref[...] = jnp.concatenate(
            [x, jnp.zeros((x.shape[0], CP - c), jnp.float32)], axis=1
        )
    sq_ref[...] = jnp.sum(x * x, axis=1, keepdims=True)


def _pad_sq(x):
    n, c = x.shape
    blk = 256
    return pl.pallas_call(
        _pad_sq_body,
        grid=(n // blk,),
        in_specs=[pl.BlockSpec((blk, c), lambda i: (i, 0))],
        out_specs=[
            pl.BlockSpec((blk, CP), lambda i: (i, 0)),
            pl.BlockSpec((blk, 1), lambda i: (i, 0)),
        ],
        out_shape=[
            jax.ShapeDtypeStruct((n, CP), jnp.float32),
            jax.ShapeDtypeStruct((n, 1), jnp.float32),
        ],
    )(x)


KNN_BLK = 128


def _knn_body(xr_ref, x_ref, sqr_ref, sqt_ref, bfr_ref, bft_ref, nb_ref, d_ref):
    i = pl.program_id(0)
    xr = xr_ref[...]
    xall = x_ref[...]
    d = sqr_ref[...] - 2.0 * lax.dot_general(
        xr.astype(jnp.bfloat16), xall.astype(jnp.bfloat16),
        (((1,), (1,)), ((), ())), preferred_element_type=jnp.float32,
    ) + sqt_ref[...]
    col = lax.broadcasted_iota(jnp.int32, (KNN_BLK, N), 1).astype(jnp.float32)
    row = lax.broadcasted_iota(jnp.int32, (KNN_BLK, N), 0).astype(jnp.float32) + jnp.float32(KNN_BLK) * i.astype(jnp.float32)
    mask = (bfr_ref[...] != bft_ref[...]) | (row == col)
    d_ref[...] = jnp.where(mask, INF, d)

    lm = jnp.full((KNN_BLK, 1), -INF, jnp.float32)
    li = jnp.full((KNN_BLK, 1), -1.0, jnp.float32)
    tcol = lax.broadcasted_iota(jnp.int32, (KNN_BLK, K), 1).astype(jnp.float32)
    nb = jnp.zeros((KNN_BLK, K), jnp.float32)
    for t in range(K):
        dd = d_ref[...]
        cand = (dd > lm) | ((dd == lm) & (col > li))
        dm = jnp.where(cand, dd, INF)
        m = jnp.min(dm, axis=1, keepdims=True)
        idx = jnp.min(jnp.where(cand & (dd <= m), col, NF), axis=1, keepdims=True)
        nb = nb + jnp.where(tcol == jnp.float32(t), jnp.broadcast_to(idx, (KNN_BLK, K)), 0.0)
        lm = m
        li = idx
    nb_ref[...] = nb.astype(jnp.int32)


def _knn(x, sq, sqt, bfr, bft):
    n, cin = x.shape
    grid = n // KNN_BLK
    return pl.pallas_call(
        _knn_body,
        grid=(grid,),
        in_specs=[
            pl.BlockSpec((KNN_BLK, cin), lambda i: (i, 0)),
            pl.BlockSpec((n, cin), lambda i: (0, 0)),
            pl.BlockSpec((KNN_BLK, 1), lambda i: (i, 0)),
            pl.BlockSpec((1, n), lambda i: (0, 0)),
            pl.BlockSpec((KNN_BLK, 1), lambda i: (i, 0)),
            pl.BlockSpec((1, n), lambda i: (0, 0)),
        ],
        out_specs=pl.BlockSpec((KNN_BLK, K), lambda i: (i, 0)),
        out_shape=jax.ShapeDtypeStruct((n, K), jnp.int32),
        scratch_shapes=[pltpu.VMEM((KNN_BLK, N), jnp.float32)],
    )(x, x, sq, sqt, bfr, bft)


SC_WORKERS = 32
GBLK = 64


def _sc_gather(idx_flat, xp):
    m = idx_flat.shape[0]
    rows_per_w = m // SC_WORKERS
    n_chunks = rows_per_w // GBLK
    mesh = plsc.VectorSubcoreMesh(
        core_axis_name="c", subcore_axis_name="s", num_cores=2, num_subcores=16
    )

    @functools.partial(
        pl.kernel,
        out_type=jax.ShapeDtypeStruct((m, CP), jnp.float32),
        mesh=mesh,
        scratch_types=[
            pltpu.VMEM((GBLK,), jnp.int32),
            pltpu.VMEM((GBLK, CP), jnp.float32),
            pltpu.SemaphoreType.DMA,
        ],
    )
    def gk(idx_hbm, x_hbm, out_hbm, idx_v, buf, sem):
        wid = lax.axis_index("s") * 2 + lax.axis_index("c")
        base_w = wid * rows_per_w

        def chunk_body(ci, _):
            base = base_w + ci * GBLK
            pltpu.sync_copy(idx_hbm.at[pl.ds(base, GBLK)], idx_v)
            pltpu.async_copy(x_hbm.at[idx_v], buf, sem).wait()
            pltpu.sync_copy(buf, out_hbm.at[pl.ds(base, GBLK)])
            return 0

        lax.fori_loop(0, n_chunks, chunk_body, 0, unroll=False)

    return gk(idx_flat, xp)


EBLK = 256


def _edge_mm_body(x_ref, xj_ref, w_ref, h_ref, s_ref):
    k = pl.program_id(0)
    i = pl.program_id(1)

    @pl.when((k == 0) & (i == 0))
    def _():
        s_ref[...] = jnp.zeros_like(s_ref)

    xi = x_ref[...]
    c = xi.shape[1]
    xj = xj_ref[...].reshape(EBLK, CP)[:, :c]
    e = jnp.concatenate([xi, xj - xi], axis=1)
    h = jnp.dot(e.astype(jnp.bfloat16), w_ref[...].astype(jnp.bfloat16),
                preferred_element_type=jnp.float32)
    co = h.shape[1]
    h_ref[...] = h.reshape(1, EBLK, co)
    p1 = jnp.sum(h, axis=0, keepdims=True)
    p2 = jnp.sum(h * h, axis=0, keepdims=True)
    rowi = lax.broadcasted_iota(jnp.int32, (8, co), 0)
    upd = jnp.where(rowi == 0, jnp.broadcast_to(p1, (8, co)), 0.0) + jnp.where(
        rowi == 1, jnp.broadcast_to(p2, (8, co)), 0.0
    )
    s_ref[...] = s_ref[...] + upd


def _edge_mm(x, xj3, w):
    n, c = x.shape
    co = w.shape[1]
    return pl.pallas_call(
        _edge_mm_body,
        grid=(K, n // EBLK),
        in_specs=[
            pl.BlockSpec((EBLK, c), lambda k, i: (i, 0)),
            pl.BlockSpec((1, EBLK, CP), lambda k, i: (k, i, 0)),
            pl.BlockSpec((2 * c, co), lambda k, i: (0, 0)),
        ],
        out_specs=[
            pl.BlockSpec((1, EBLK, co), lambda k, i: (k, i, 0)),
            pl.BlockSpec((8, co), lambda k, i: (0, 0)),
        ],
        out_shape=[
            jax.ShapeDtypeStruct((K, n, co), jnp.float32),
            jax.ShapeDtypeStruct((8, co), jnp.float32),
        ],
    )(x, xj3, w)


def _lrelu(x):
    return jnp.where(x >= 0.0, x, 0.2 * x)


def _norm_max_body(h_ref, s_ref, gm_ref, bt_ref, o_ref):
    k = pl.program_id(1)
    co = gm_ref.shape[1]
    denom = jnp.float32(N * K)
    s1 = s_ref[0:1, :]
    s2 = s_ref[1:2, :]
    m = s1 / denom
    var = s2 / denom - m * m
    h = h_ref[...].reshape(EBLK, co)
    y = _lrelu(gm_ref[...] * (h - m) / jnp.sqrt(var + 1e-5) + bt_ref[...])

    @pl.when(k == 0)
    def _():
        o_ref[...] = y

    @pl.when(k > 0)
    def _():
        o_ref[...] = jnp.maximum(o_ref[...], y)


def _norm_max(h3, s, gm, bt):
    _, n, co = h3.shape
    return pl.pallas_call(
        _norm_max_body,
        grid=(n // EBLK, K),
        in_specs=[
            pl.BlockSpec((1, EBLK, co), lambda i, k: (k, i, 0)),
            pl.BlockSpec((8, co), lambda i, k: (0, 0)),
            pl.BlockSpec((1, co), lambda i, k: (0, 0)),
            pl.BlockSpec((1, co), lambda i, k: (0, 0)),
        ],
        out_specs=pl.BlockSpec((EBLK, co), lambda i, k: (i, 0)),
        out_shape=jax.ShapeDtypeStruct((n, co), jnp.float32),
    )(h3, s, gm, bt)


def _feat_body(x1_ref, x2_ref, x3_ref, wf_ref, h_ref, s_ref):
    i = pl.program_id(0)

    @pl.when(i == 0)
    def _():
        s_ref[...] = jnp.zeros_like(s_ref)

    xc = jnp.concatenate([x1_ref[...], x2_ref[...], x3_ref[...]], axis=1)
    h = jnp.dot(xc.astype(jnp.bfloat16), wf_ref[...].astype(jnp.bfloat16),
                preferred_element_type=jnp.float32)
    h_ref[...] = h
    p1 = jnp.sum(h, axis=0, keepdims=True)
    p2 = jnp.sum(h * h, axis=0, keepdims=True)
    cf = h.shape[1]
    rowi = lax.broadcasted_iota(jnp.int32, (8, cf), 0)
    upd = jnp.where(rowi == 0, jnp.broadcast_to(p1, (8, cf)), 0.0) + jnp.where(
        rowi == 1, jnp.broadcast_to(p2, (8, cf)), 0.0
    )
    s_ref[...] = s_ref[...] + upd


def _feat(x1, x2, x3, wf):
    cf = wf.shape[1]
    blk = 256
    return pl.pallas_call(
        _feat_body,
        grid=(N // blk,),
        in_specs=[
            pl.BlockSpec((blk, 64), lambda i: (i, 0)),
            pl.BlockSpec((blk, 64), lambda i: (i, 0)),
            pl.BlockSpec((blk, 128), lambda i: (i, 0)),
            pl.BlockSpec((256, cf), lambda i: (0, 0)),
        ],
        out_specs=[
            pl.BlockSpec((blk, cf), lambda i: (i, 0)),
            pl.BlockSpec((8, cf), lambda i: (0, 0)),
        ],
        out_shape=[
            jax.ShapeDtypeStruct((N, cf), jnp.float32),
            jax.ShapeDtypeStruct((8, cf), jnp.float32),
        ],
    )(x1, x2, x3, wf)


def _pool_body(h_ref, s_ref, gm_ref, bt_ref, bfr_ref, o_ref):
    i = pl.program_id(0)
    cf = h_ref.shape[1]

    @pl.when(i == 0)
    def _():
        o_ref[...] = jnp.full_like(o_ref, -INF)

    m = s_ref[0:1, :] / NF
    var = s_ref[1:2, :] / NF - (s_ref[0:1, :] / NF) * (s_ref[0:1, :] / NF)
    xf = _lrelu(gm_ref[...] * (h_ref[...] - m) / jnp.sqrt(var + 1e-5) + bt_ref[...])
    bf = bfr_ref[...]
    rowi = lax.broadcasted_iota(jnp.int32, (NSEG, cf), 0)
    upd = jnp.full((NSEG, cf), -INF, jnp.float32)
    for seg in range(NSEG):
        vmask = jnp.where(bf == jnp.float32(seg), xf, -INF)
        mxs = jnp.max(vmask, axis=0, keepdims=True)
        upd = jnp.where(rowi == seg, jnp.broadcast_to(mxs, (NSEG, cf)), upd)
    o_ref[...] = jnp.maximum(o_ref[...], upd)


def _pool(h, s, gm, bt, bfr):
    cf = h.shape[1]
    blk = 256
    return pl.pallas_call(
        _pool_body,
        grid=(N // blk,),
        in_specs=[
            pl.BlockSpec((blk, cf), lambda i: (i, 0)),
            pl.BlockSpec((8, cf), lambda i: (0, 0)),
            pl.BlockSpec((1, cf), lambda i: (0, 0)),
            pl.BlockSpec((1, cf), lambda i: (0, 0)),
            pl.BlockSpec((blk, 1), lambda i: (i, 0)),
        ],
        out_specs=pl.BlockSpec((NSEG, cf), lambda i: (0, 0)),
        out_shape=jax.ShapeDtypeStruct((NSEG, cf), jnp.float32),
    )(h, s, gm, bt, bfr)


def _bn_rows(h, g, b):
    m = jnp.mean(h, axis=0, keepdims=True)
    d = h - m
    v = jnp.mean(d * d, axis=0, keepdims=True)
    return g * d / jnp.sqrt(v + 1e-5) + b


def _cls_body(gp_ref, w1_ref, b1_ref, g1_ref, bb1_ref, w2_ref, b2_ref, g2_ref,
              bb2_ref, w3_ref, b3_ref, o_ref):
    def _bdot(a, b):
        return jnp.dot(a.astype(jnp.bfloat16), b.astype(jnp.bfloat16),
                       preferred_element_type=jnp.float32)

    h = _bdot(gp_ref[...], w1_ref[...]) + b1_ref[...]
    h = _lrelu(_bn_rows(h, g1_ref[...], bb1_ref[...]))
    h = _bdot(h, w2_ref[...]) + b2_ref[...]
    h = _lrelu(_bn_rows(h, g2_ref[...], bb2_ref[...]))
    o_ref[...] = _bdot(h, w3_ref[...]) + b3_ref[...]


def _cls(gp, w1, b1, g1, bb1, w2, b2, g2, bb2, w3, b3):
    return pl.pallas_call(
        _cls_body,
        out_shape=jax.ShapeDtypeStruct((NSEG, 40), jnp.float32),
    )(gp, w1, b1, g1, bb1, w2, b2, g2, bb2, w3, b3)


def _edge_conv(x, bfr, bft, w, g, b):
    xp, sq = _pad_sq(x)
    nb = _knn(x, sq, sq.reshape(1, N), bfr, bft)
    idx = nb.T.reshape(N * K)
    xj = _sc_gather(idx, xp)
    h3, s = _edge_mm(x, xj.reshape(K, N, CP), w)
    return _norm_max(h3, s, g.reshape(1, -1), b.reshape(1, -1))


def kernel(pos, batch, W1, g1, b1, W2, g2, b2, W3, g3, b3, Wf, gf, bf,
           Wc1, bc1, gcl1, bcl1, Wc2, bc2, gcl2, bcl2, Wc3, bc3):
    bfr = batch.astype(jnp.float32).reshape(N, 1)
    bft = batch.astype(jnp.float32).reshape(1, N)
    x1 = _edge_conv(pos, bfr, bft, W1, g1, b1)
    x2 = _edge_conv(x1, bfr, bft, W2, g2, b2)
    x3 = _edge_conv(x2, bfr, bft, W3, g3, b3)
    h, sf = _feat(x1, x2, x3, Wf)
    gp = _pool(h, sf, gf.reshape(1, -1), bf.reshape(1, -1), bfr)
    return _cls(gp, Wc1, bc1.reshape(1, -1), gcl1.reshape(1, -1), bcl1.reshape(1, -1),
                Wc2, bc2.reshape(1, -1), gcl2.reshape(1, -1), bcl2.reshape(1, -1),
                Wc3, bc3.reshape(1, -1))


# speedup vs baseline: 2.9707x; 1.0877x over previous
import functools

import jax
import jax.numpy as jnp
from jax import lax
from jax.experimental import pallas as pl
from jax.experimental.pallas import tpu as pltpu
from jax.experimental.pallas import tpu_sc as plsc

N = 8192
K = 20
NSEG = 8
NF = float(N)
INF = jnp.inf
CP = 128


def _pad_sq_body(x_ref, xp_ref, sq_ref):
    x = x_ref[...]
    c = x.shape[1]
    if c == CP:
        xp_ref[...] = x
    else:
        xp_ref[...] = jnp.concatenate(
            [x, jnp.zeros((x.shape[0], CP - c), jnp.float32)], axis=1
        )
    sq_ref[...] = jnp.sum(x * x, axis=1, keepdims=True)


def _pad_sq(x):
    n, c = x.shape
    blk = 256
    return pl.pallas_call(
        _pad_sq_body,
        grid=(n // blk,),
        in_specs=[pl.BlockSpec((blk, c), lambda i: (i, 0))],
        out_specs=[
            pl.BlockSpec((blk, CP), lambda i: (i, 0)),
            pl.BlockSpec((blk, 1), lambda i: (i, 0)),
        ],
        out_shape=[
            jax.ShapeDtypeStruct((n, CP), jnp.float32),
            jax.ShapeDtypeStruct((n, 1), jnp.float32),
        ],
    )(x)


KNN_BLK = 256


def _knn_body(xr_ref, x_ref, sqr_ref, sqt_ref, bfr_ref, bft_ref, nb_ref, d_ref):
    i = pl.program_id(0)
    xr = xr_ref[...]
    xall = x_ref[...]
    d = sqr_ref[...] - 2.0 * lax.dot_general(
        xr.astype(jnp.bfloat16), xall.astype(jnp.bfloat16),
        (((1,), (1,)), ((), ())), preferred_element_type=jnp.float32,
    ) + sqt_ref[...]
    col = lax.broadcasted_iota(jnp.int32, (KNN_BLK, N), 1).astype(jnp.float32)
    row = lax.broadcasted_iota(jnp.int32, (KNN_BLK, N), 0).astype(jnp.float32) + jnp.float32(KNN_BLK) * i.astype(jnp.float32)
    mask = (bfr_ref[...] != bft_ref[...]) | (row == col)
    d_ref[...] = jnp.where(mask, INF, d)

    lm = jnp.full((KNN_BLK, 1), -INF, jnp.float32)
    li = jnp.full((KNN_BLK, 1), -1.0, jnp.float32)
    tcol = lax.broadcasted_iota(jnp.int32, (KNN_BLK, K), 1).astype(jnp.float32)
    nb = jnp.zeros((KNN_BLK, K), jnp.float32)
    for t in range(K):
        dd = d_ref[...]
        cand = (dd > lm) | ((dd == lm) & (col > li))
        dm = jnp.where(cand, dd, INF)
        m = jnp.min(dm, axis=1, keepdims=True)
        idx = jnp.min(jnp.where(cand & (dd <= m), col, NF), axis=1, keepdims=True)
        nb = nb + jnp.where(tcol == jnp.float32(t), jnp.broadcast_to(idx, (KNN_BLK, K)), 0.0)
        lm = m
        li = idx
    nb_ref[...] = nb.astype(jnp.int32)


def _knn(x, sq, sqt, bfr, bft):
    n, cin = x.shape
    grid = n // KNN_BLK
    return pl.pallas_call(
        _knn_body,
        grid=(grid,),
        in_specs=[
            pl.BlockSpec((KNN_BLK, cin), lambda i: (i, 0)),
            pl.BlockSpec((n, cin), lambda i: (0, 0)),
            pl.BlockSpec((KNN_BLK, 1), lambda i: (i, 0)),
            pl.BlockSpec((1, n), lambda i: (0, 0)),
            pl.BlockSpec((KNN_BLK, 1), lambda i: (i, 0)),
            pl.BlockSpec((1, n), lambda i: (0, 0)),
        ],
        out_specs=pl.BlockSpec((KNN_BLK, K), lambda i: (i, 0)),
        out_shape=jax.ShapeDtypeStruct((n, K), jnp.int32),
        scratch_shapes=[pltpu.VMEM((KNN_BLK, N), jnp.float32)],
    )(x, x, sq, sqt, bfr, bft)


SC_WORKERS = 32
GBLK = 64


def _sc_gather(idx_flat, xp):
    m = idx_flat.shape[0]
    rows_per_w = m // SC_WORKERS
    n_chunks = rows_per_w // GBLK
    mesh = plsc.VectorSubcoreMesh(
        core_axis_name="c", subcore_axis_name="s", num_cores=2, num_subcores=16
    )

    @functools.partial(
        pl.kernel,
        out_type=jax.ShapeDtypeStruct((m, CP), jnp.float32),
        mesh=mesh,
        scratch_types=[
            pltpu.VMEM((GBLK,), jnp.int32),
            pltpu.VMEM((GBLK, CP), jnp.float32),
            pltpu.SemaphoreType.DMA,
        ],
    )
    def gk(idx_hbm, x_hbm, out_hbm, idx_v, buf, sem):
        wid = lax.axis_index("s") * 2 + lax.axis_index("c")
        base_w = wid * rows_per_w

        def chunk_body(ci, _):
            base = base_w + ci * GBLK
            pltpu.sync_copy(idx_hbm.at[pl.ds(base, GBLK)], idx_v)
            pltpu.async_copy(x_hbm.at[idx_v], buf, sem).wait()
            pltpu.sync_copy(buf, out_hbm.at[pl.ds(base, GBLK)])
            return 0

        lax.fori_loop(0, n_chunks, chunk_body, 0, unroll=False)

    return gk(idx_flat, xp)


EBLK = 256


def _edge_mm_body(x_ref, xj_ref, w_ref, h_ref, s_ref):
    k = pl.program_id(0)
    i = pl.program_id(1)

    @pl.when((k == 0) & (i == 0))
    def _():
        s_ref[...] = jnp.zeros_like(s_ref)

    xi = x_ref[...]
    c = xi.shape[1]
    xj = xj_ref[...].reshape(EBLK, CP)[:, :c]
    e = jnp.concatenate([xi, xj - xi], axis=1)
    h = jnp.dot(e.astype(jnp.bfloat16), w_ref[...].astype(jnp.bfloat16),
                preferred_element_type=jnp.float32)
    co = h.shape[1]
    h_ref[...] = h.reshape(1, EBLK, co)
    p1 = jnp.sum(h, axis=0, keepdims=True)
    p2 = jnp.sum(h * h, axis=0, keepdims=True)
    rowi = lax.broadcasted_iota(jnp.int32, (8, co), 0)
    upd = jnp.where(rowi == 0, jnp.broadcast_to(p1, (8, co)), 0.0) + jnp.where(
        rowi == 1, jnp.broadcast_to(p2, (8, co)), 0.0
    )
    s_ref[...] = s_ref[...] + upd


def _edge_mm(x, xj3, w):
    n, c = x.shape
    co = w.shape[1]
    return pl.pallas_call(
        _edge_mm_body,
        grid=(K, n // EBLK),
        in_specs=[
            pl.BlockSpec((EBLK, c), lambda k, i: (i, 0)),
            pl.BlockSpec((1, EBLK, CP), lambda k, i: (k, i, 0)),
            pl.BlockSpec((2 * c, co), lambda k, i: (0, 0)),
        ],
        out_specs=[
            pl.BlockSpec((1, EBLK, co), lambda k, i: (k, i, 0)),
            pl.BlockSpec((8, co), lambda k, i: (0, 0)),
        ],
        out_shape=[
            jax.ShapeDtypeStruct((K, n, co), jnp.float32),
            jax.ShapeDtypeStruct((8, co), jnp.float32),
        ],
    )(x, xj3, w)


def _lrelu(x):
    return jnp.where(x >= 0.0, x, 0.2 * x)


def _norm_max_body(h_ref, s_ref, gm_ref, bt_ref, o_ref):
    k = pl.program_id(1)
    co = gm_ref.shape[1]
    denom = jnp.float32(N * K)
    s1 = s_ref[0:1, :]
    s2 = s_ref[1:2, :]
    m = s1 / denom
    var = s2 / denom - m * m
    h = h_ref[...].reshape(EBLK, co)
    y = _lrelu(gm_ref[...] * (h - m) / jnp.sqrt(var + 1e-5) + bt_ref[...])

    @pl.when(k == 0)
    def _():
        o_ref[...] = y

    @pl.when(k > 0)
    def _():
        o_ref[...] = jnp.maximum(o_ref[...], y)


def _norm_max(h3, s, gm, bt):
    _, n, co = h3.shape
    return pl.pallas_call(
        _norm_max_body,
        grid=(n // EBLK, K),
        in_specs=[
            pl.BlockSpec((1, EBLK, co), lambda i, k: (k, i, 0)),
            pl.BlockSpec((8, co), lambda i, k: (0, 0)),
            pl.BlockSpec((1, co), lambda i, k: (0, 0)),
            pl.BlockSpec((1, co), lambda i, k: (0, 0)),
        ],
        out_specs=pl.BlockSpec((EBLK, co), lambda i, k: (i, 0)),
        out_shape=jax.ShapeDtypeStruct((n, co), jnp.float32),
    )(h3, s, gm, bt)


def _feat_body(x1_ref, x2_ref, x3_ref, wf_ref, h_ref, s_ref):
    i = pl.program_id(0)

    @pl.when(i == 0)
    def _():
        s_ref[...] = jnp.zeros_like(s_ref)

    xc = jnp.concatenate([x1_ref[...], x2_ref[...], x3_ref[...]], axis=1)
    h = jnp.dot(xc.astype(jnp.bfloat16), wf_ref[...].astype(jnp.bfloat16),
                preferred_element_type=jnp.float32)
    h_ref[...] = h
    p1 = jnp.sum(h, axis=0, keepdims=True)
    p2 = jnp.sum(h * h, axis=0, keepdims=True)
    cf = h.shape[1]
    rowi = lax.broadcasted_iota(jnp.int32, (8, cf), 0)
    upd = jnp.where(rowi == 0, jnp.broadcast_to(p1, (8, cf)), 0.0) + jnp.where(
        rowi == 1, jnp.broadcast_to(p2, (8, cf)), 0.0
    )
    s_ref[...] = s_ref[...] + upd


def _feat(x1, x2, x3, wf):
    cf = wf.shape[1]
    blk = 256
    return pl.pallas_call(
        _feat_body,
        grid=(N // blk,),
        in_specs=[
            pl.BlockSpec((blk, 64), lambda i: (i, 0)),
            pl.BlockSpec((blk, 64), lambda i: (i, 0)),
            pl.BlockSpec((blk, 128), lambda i: (i, 0)),
            pl.BlockSpec((256, cf), lambda i: (0, 0)),
        ],
        out_specs=[
            pl.BlockSpec((blk, cf), lambda i: (i, 0)),
            pl.BlockSpec((8, cf), lambda i: (0, 0)),
        ],
        out_shape=[
            jax.ShapeDtypeStruct((N, cf), jnp.float32),
            jax.ShapeDtypeStruct((8, cf), jnp.float32),
        ],
    )(x1, x2, x3, wf)


def _pool_body(h_ref, s_ref, gm_ref, bt_ref, bfr_ref, o_ref):
    i = pl.program_id(0)
    cf = h_ref.shape[1]

    @pl.when(i == 0)
    def _():
        o_ref[...] = jnp.full_like(o_ref, -INF)

    m = s_ref[0:1, :] / NF
    var = s_ref[1:2, :] / NF - (s_ref[0:1, :] / NF) * (s_ref[0:1, :] / NF)
    xf = _lrelu(gm_ref[...] * (h_ref[...] - m) / jnp.sqrt(var + 1e-5) + bt_ref[...])
    bf = bfr_ref[...]
    rowi = lax.broadcasted_iota(jnp.int32, (NSEG, cf), 0)
    upd = jnp.full((NSEG, cf), -INF, jnp.float32)
    for seg in range(NSEG):
        vmask = jnp.where(bf == jnp.float32(seg), xf, -INF)
        mxs = jnp.max(vmask, axis=0, keepdims=True)
        upd = jnp.where(rowi == seg, jnp.broadcast_to(mxs, (NSEG, cf)), upd)
    o_ref[...] = jnp.maximum(o_ref[...], upd)


def _pool(h, s, gm, bt, bfr):
    cf = h.shape[1]
    blk = 256
    return pl.pallas_call(
        _pool_body,
        grid=(N // blk,),
        in_specs=[
            pl.BlockSpec((blk, cf), lambda i: (i, 0)),
            pl.BlockSpec((8, cf), lambda i: (0, 0)),
            pl.BlockSpec((1, cf), lambda i: (0, 0)),
            pl.BlockSpec((1, cf), lambda i: (0, 0)),
            pl.BlockSpec((blk, 1), lambda i: (i, 0)),
        ],
        out_specs=pl.BlockSpec((NSEG, cf), lambda i: (0, 0)),
        out_shape=jax.ShapeDtypeStruct((NSEG, cf), jnp.float32),
    )(h, s, gm, bt, bfr)


def _bn_rows(h, g, b):
    m = jnp.mean(h, axis=0, keepdims=True)
    d = h - m
    v = jnp.mean(d * d, axis=0, keepdims=True)
    return g * d / jnp.sqrt(v + 1e-5) + b


def _cls_body(gp_ref, w1_ref, b1_ref, g1_ref, bb1_ref, w2_ref, b2_ref, g2_ref,
              bb2_ref, w3_ref, b3_ref, o_ref):
    def _bdot(a, b):
        return jnp.dot(a.astype(jnp.bfloat16), b.astype(jnp.bfloat16),
                       preferred_element_type=jnp.float32)

    h = _bdot(gp_ref[...], w1_ref[...]) + b1_ref[...]
    h = _lrelu(_bn_rows(h, g1_ref[...], bb1_ref[...]))
    h = _bdot(h, w2_ref[...]) + b2_ref[...]
    h = _lrelu(_bn_rows(h, g2_ref[...], bb2_ref[...]))
    o_ref[...] = _bdot(h, w3_ref[...]) + b3_ref[...]


def _cls(gp, w1, b1, g1, bb1, w2, b2, g2, bb2, w3, b3):
    return pl.pallas_call(
        _cls_body,
        out_shape=jax.ShapeDtypeStruct((NSEG, 40), jnp.float32),
    )(gp, w1, b1, g1, bb1, w2, b2, g2, bb2, w3, b3)


def _edge_conv(x, bfr, bft, w, g, b):
    xp, sq = _pad_sq(x)
    nb = _knn(x, sq, sq.reshape(1, N), bfr, bft)
    idx = nb.T.reshape(N * K)
    xj = _sc_gather(idx, xp)
    h3, s = _edge_mm(x, xj.reshape(K, N, CP), w)
    return _norm_max(h3, s, g.reshape(1, -1), b.reshape(1, -1))


def kernel(pos, batch, W1, g1, b1, W2, g2, b2, W3, g3, b3, Wf, gf, bf,
           Wc1, bc1, gcl1, bcl1, Wc2, bc2, gcl2, bcl2, Wc3, bc3):
    bfr = batch.astype(jnp.float32).reshape(N, 1)
    bft = batch.astype(jnp.float32).reshape(1, N)
    x1 = _edge_conv(pos, bfr, bft, W1, g1, b1)
    x2 = _edge_conv(x1, bfr, bft, W2, g2, b2)
    x3 = _edge_conv(x2, bfr, bft, W3, g3, b3)
    h, sf = _feat(x1, x2, x3, Wf)
    gp = _pool(h, sf, gf.reshape(1, -1), bf.reshape(1, -1), bfr)
    return _cls(gp, Wc1, bc1.reshape(1, -1), gcl1.reshape(1, -1), bcl1.reshape(1, -1),
                Wc2, bc2.reshape(1, -1), gcl2.reshape(1, -1), bcl2.reshape(1, -1),
                Wc3, bc3.reshape(1, -1))
